# baseline (speedup 1.0000x reference)
import jax
import jax.numpy as jnp
from jax import lax
from jax.experimental import pallas as pl
from jax.experimental.pallas import tpu as pltpu
from jax.experimental.pallas import tpu_sc as plsc

HOTN = 32768
HASH_SIZE = 500000
EMB_DIM = 32
BATCH = 16384

_NC = 2
_NS = 16
_NW = _NC * _NS
_BPW = BATCH // _NW
_NVEC = _BPW // 16
_CH = 128
_NCH = _BPW // _CH


def _repack_body(src_ref, dst_ref):
    x = src_ref[...]
    y = x.reshape(32, 128, 4)
    dst_ref[...] = y.transpose(1, 2, 0).reshape(128, 128)


def _repack(wt, n_rows):
    grid = (n_rows + 511) // 512
    return pl.pallas_call(
        _repack_body,
        grid=(grid,),
        in_specs=[pl.BlockSpec((32, 512), lambda b: (0, b))],
        out_specs=pl.BlockSpec((128, 128), lambda b: (b, 0)),
        out_shape=jax.ShapeDtypeStruct((n_rows // 4, 128), jnp.float32),
    )(wt)


def _sc_body(inp_hbm, wh_hbm, whash_hbm, out_hbm,
             raw_v, gh_v, gc_v, offh_v, offc_v, maskf_v,
             hot_b, cold_b, out_b, sem):
    wid = lax.axis_index("s") * _NC + lax.axis_index("c")
    base = wid * _BPW

    pltpu.sync_copy(inp_hbm.at[pl.ds(base, _BPW)], raw_v)

    for i in range(_NVEC):
        v = raw_v[pl.ds(i * 16, 16)]
        s = (v & 31) + ((v >> 5) & 31) + ((v >> 10) & 31) + ((v >> 15) & 31)
        s = (s & 31) + (s >> 5)
        hot = jnp.logical_or(s == 0, s == 31)
        maskf_v[pl.ds(i * 16, 16)] = jnp.where(hot, 1.0, 0.0).astype(jnp.float32)
        rh = v & (HOTN - 1)
        rc = jnp.where(v >= HASH_SIZE, v - HASH_SIZE, v)
        gh_v[i // 8, pl.ds((i % 8) * 16, 16)] = rh >> 2
        gc_v[i // 8, pl.ds((i % 8) * 16, 16)] = rc >> 2
        offh_v[pl.ds(i * 16, 16)] = (rh & 3) << 5
        offc_v[pl.ds(i * 16, 16)] = (rc & 3) << 5

    for ch in range(_NCH):
        c1 = pltpu.async_copy(wh_hbm.at[gh_v.at[ch]], hot_b, sem)
        c2 = pltpu.async_copy(whash_hbm.at[gc_v.at[ch]], cold_b, sem)
        c1.wait()
        c2.wait()

        def blend(blk, _):
            b16 = ch * _CH + blk * 16
            offh16 = offh_v[pl.ds(b16, 16)]
            offc16 = offc_v[pl.ds(b16, 16)]
            m16 = maskf_v[pl.ds(b16, 16)]
            for jj in range(16):
                oh = offh16[jj]
                oc = offc16[jj]
                m = m16[jj]
                il = blk * 16 + jj
                orow = blk * 4 + (jj >> 2)
                ocol = (jj & 3) * 32
                for c0 in (0, 16):
                    h = hot_b[il, pl.ds(oh + c0, 16)]
                    g = cold_b[il, pl.ds(oc + c0, 16)]
                    out_b[orow, pl.ds(ocol + c0, 16)] = g + m * (h - g)
            return 0

        lax.fori_loop(0, _CH // 16, blend, 0)
        pltpu.sync_copy(out_b, out_hbm.at[pl.ds(wid * 128 + ch * 32, 32)])


@jax.jit
def _run(inp, wt_h, wt_hash):
    wh = _repack(wt_h, HOTN)
    whash = _repack(wt_hash, HASH_SIZE)
    mesh = plsc.VectorSubcoreMesh(core_axis_name="c", subcore_axis_name="s")
    f = pl.kernel(
        _sc_body,
        out_type=jax.ShapeDtypeStruct((BATCH // 4, 128), jnp.float32),
        mesh=mesh,
        compiler_params=pltpu.CompilerParams(use_tc_tiling_on_sc=True),
        scratch_types=[
            pltpu.VMEM((_BPW,), jnp.int32),
            pltpu.VMEM((_NCH, _CH), jnp.int32),
            pltpu.VMEM((_NCH, _CH), jnp.int32),
            pltpu.VMEM((_BPW,), jnp.int32),
            pltpu.VMEM((_BPW,), jnp.int32),
            pltpu.VMEM((_BPW,), jnp.float32),
            pltpu.VMEM((_CH, 128), jnp.float32),
            pltpu.VMEM((_CH, 128), jnp.float32),
            pltpu.VMEM((32, 128), jnp.float32),
            pltpu.SemaphoreType.DMA,
        ],
    )
    return f(inp, wh, whash)


def kernel(input, offsets, weight_h, weight_hash):
    del offsets
    out = _run(input.astype(jnp.int32), weight_h.T, weight_hash.T)
    return out.reshape(BATCH, EMB_DIM)


# speedup vs baseline: 3.3144x; 3.3144x over previous
import jax
import jax.numpy as jnp
import numpy as np
from jax import lax
from jax.experimental import pallas as pl
from jax.experimental.pallas import tpu as pltpu
from jax.experimental.pallas import tpu_sc as plsc

HOTN = 32768
HASH_SIZE = 500000
EMB_DIM = 32
BATCH = 16384

_NC = 2
_NS = 16
_NW = _NC * _NS
_BPW = BATCH // _NW
_NVEC = _BPW // 16
_CH = 128
_NCH = _BPW // _CH

_RC = 512
_NFULL = HASH_SIZE // _RC
_TAIL0 = _NFULL * _RC
_TAILN = HASH_SIZE - _TAIL0
_KMAX = (_NFULL + _NW - 1) // _NW


def _repack_body(wth_hbm, wtc_hbm, tailp_hbm, ph_hbm, pc_hbm,
                 src_b, dst_b, sem):
    wid = lax.axis_index("s") * _NC + lax.axis_index("c")
    lane = jnp.arange(16, dtype=jnp.int32)
    rowpat = lane >> 2
    colpat = (lane & 3) << 5

    def chunk(src_hbm, dst_hbm, c0, csz):
        pltpu.sync_copy(src_hbm.at[:, pl.ds(c0, csz)],
                        src_b.at[:, pl.ds(0, csz)])

        def v_iter(v, _):
            rowi = rowpat + v * 4
            for d in range(32):
                x = src_b[d, pl.ds(v * 16, 16)]
                plsc.store_scatter(dst_b, [rowi, colpat + d], x)
            return 0

        lax.fori_loop(0, csz // 16, v_iter, 0)
        g0 = pl.multiple_of(c0 // 4, 128)
        pltpu.sync_copy(dst_b.at[pl.ds(0, csz // 4)],
                        dst_hbm.at[pl.ds(g0, csz // 4)])

    for k in range(2):
        cid = wid + k * _NW
        chunk(wth_hbm, ph_hbm, pl.multiple_of(cid * _RC, _RC), _RC)

    def hash_round(k, _):
        cid = wid + k * _NW

        @pl.when(cid < _NFULL)
        def _():
            chunk(wtc_hbm, pc_hbm, pl.multiple_of(cid * _RC, _RC), _RC)
        return 0

    lax.fori_loop(0, _KMAX, hash_round, 0)

    @pl.when(wid == 1)
    def _():
        chunk(wtc_hbm, pc_hbm, _TAIL0, 256)

    @pl.when(wid == 2)
    def _():
        pltpu.sync_copy(tailp_hbm, dst_b.at[pl.ds(0, 8)])
        pltpu.sync_copy(dst_b.at[pl.ds(0, 8)],
                        pc_hbm.at[pl.ds((HASH_SIZE - 32) // 4, 8)])


def _sc_body(inp_hbm, wh_hbm, whash_hbm, out_hbm,
             raw_v, gh_v, gc_v, offh_v, offc_v, maskf_v,
             hot_b, cold_b, out_b, sem):
    wid = lax.axis_index("s") * _NC + lax.axis_index("c")
    base = wid * _BPW

    pltpu.sync_copy(inp_hbm.at[pl.ds(base, _BPW)], raw_v)

    for i in range(_NVEC):
        v = raw_v[pl.ds(i * 16, 16)]
        s = (v & 31) + ((v >> 5) & 31) + ((v >> 10) & 31) + ((v >> 15) & 31)
        s = (s & 31) + (s >> 5)
        hot = jnp.logical_or(s == 0, s == 31)
        maskf_v[pl.ds(i * 16, 16)] = jnp.where(hot, 1.0, 0.0).astype(jnp.float32)
        rh = v & (HOTN - 1)
        rc = jnp.where(v >= HASH_SIZE, v - HASH_SIZE, v)
        gh_v[i // 8, pl.ds((i % 8) * 16, 16)] = rh >> 2
        gc_v[i // 8, pl.ds((i % 8) * 16, 16)] = rc >> 2
        offh_v[pl.ds(i * 16, 16)] = (rh & 3) << 5
        offc_v[pl.ds(i * 16, 16)] = (rc & 3) << 5

    for ch in range(_NCH):
        c1 = pltpu.async_copy(wh_hbm.at[gh_v.at[ch]], hot_b, sem)
        c2 = pltpu.async_copy(whash_hbm.at[gc_v.at[ch]], cold_b, sem)
        c1.wait()
        c2.wait()

        def blend(blk, _):
            b16 = ch * _CH + blk * 16
            offh16 = offh_v[pl.ds(b16, 16)]
            offc16 = offc_v[pl.ds(b16, 16)]
            m16 = maskf_v[pl.ds(b16, 16)]
            for jj in range(16):
                oh = offh16[jj]
                oc = offc16[jj]
                m = m16[jj]
                il = blk * 16 + jj
                orow = blk * 4 + (jj >> 2)
                ocol = (jj & 3) * 32
                for c0 in (0, 16):
                    h = hot_b[il, pl.ds(oh + c0, 16)]
                    g = cold_b[il, pl.ds(oc + c0, 16)]
                    out_b[orow, pl.ds(ocol + c0, 16)] = g + m * (h - g)
            return 0

        lax.fori_loop(0, _CH // 16, blend, 0)
        pltpu.sync_copy(out_b, out_hbm.at[pl.ds(wid * 128 + ch * 32, 32)])


@jax.jit
def _run(inp, wt_h, wt_hash):
    mesh = plsc.VectorSubcoreMesh(core_axis_name="c", subcore_axis_name="s")
    repack = pl.kernel(
        _repack_body,
        out_type=(jax.ShapeDtypeStruct((HOTN // 4, 128), jnp.float32),
                  jax.ShapeDtypeStruct((HASH_SIZE // 4, 128), jnp.float32)),
        mesh=mesh,
        compiler_params=pltpu.CompilerParams(use_tc_tiling_on_sc=True,
                                             needs_layout_passes=False),
        scratch_types=[
            pltpu.VMEM((32, _RC), jnp.float32),
            pltpu.VMEM((_RC // 4, 128), jnp.float32),
            pltpu.SemaphoreType.DMA,
        ],
    )
    tailp = wt_hash[:, HASH_SIZE - 32:].T.reshape(8, 128)
    wh, whash = repack(wt_h, wt_hash, tailp)

    lookup = pl.kernel(
        _sc_body,
        out_type=jax.ShapeDtypeStruct((BATCH // 4, 128), jnp.float32),
        mesh=mesh,
        compiler_params=pltpu.CompilerParams(use_tc_tiling_on_sc=True),
        scratch_types=[
            pltpu.VMEM((_BPW,), jnp.int32),
            pltpu.VMEM((_NCH, _CH), jnp.int32),
            pltpu.VMEM((_NCH, _CH), jnp.int32),
            pltpu.VMEM((_BPW,), jnp.int32),
            pltpu.VMEM((_BPW,), jnp.int32),
            pltpu.VMEM((_BPW,), jnp.float32),
            pltpu.VMEM((_CH, 128), jnp.float32),
            pltpu.VMEM((_CH, 128), jnp.float32),
            pltpu.VMEM((32, 128), jnp.float32),
            pltpu.SemaphoreType.DMA,
        ],
    )
    return lookup(inp, wh, whash)


def kernel(input, offsets, weight_h, weight_hash):
    del offsets
    out = _run(input.astype(jnp.int32), weight_h.T, weight_hash.T)
    return out.reshape(BATCH, EMB_DIM)


# speedup vs baseline: 6.1750x; 1.8631x over previous
import jax
import jax.numpy as jnp
from jax import lax
from jax.experimental import pallas as pl
from jax.experimental.pallas import tpu as pltpu
from jax.experimental.pallas import tpu_sc as plsc

HOTN = 32768
HASH_SIZE = 500000
EMB_DIM = 32
BATCH = 16384

_NC = 2
_NS = 16
_NW = _NC * _NS
_BPW = BATCH // _NW
_NVEC = _BPW // 16
_CH = 32
_NCH = _BPW // _CH


def _sc_body(inp_hbm, wh_hbm, whash_hbm, out_hbm,
             raw_v, sh_v, sc_v, rh_v, rc_v, maskf_v,
             hot_b, cold_b, out_b, sem):
    wid = lax.axis_index("s") * _NC + lax.axis_index("c")
    base = wid * _BPW

    pltpu.sync_copy(inp_hbm.at[pl.ds(base, _BPW)], raw_v)

    for i in range(_NVEC):
        v = raw_v[pl.ds(i * 16, 16)]
        s = (v & 31) + ((v >> 5) & 31) + ((v >> 10) & 31) + ((v >> 15) & 31)
        s = (s & 31) + (s >> 5)
        hot = jnp.logical_or(s == 0, s == 31)
        maskf_v[pl.ds(i * 16, 16)] = jnp.where(hot, 1.0, 0.0).astype(jnp.float32)
        rh = v & (HOTN - 1)
        rc = jnp.where(v >= HASH_SIZE, v - HASH_SIZE, v)
        sh_v[pl.ds(i * 16, 16)] = rh & ~7
        sc_v[pl.ds(i * 16, 16)] = rc & ~7
        rh_v[pl.ds(i * 16, 16)] = rh & 7
        rc_v[pl.ds(i * 16, 16)] = rc & 7

    def chunk(ch, _):
        b32 = ch * _CH
        sh0 = sh_v[pl.ds(b32, 16)]
        sh1 = sh_v[pl.ds(b32 + 16, 16)]
        sc0 = sc_v[pl.ds(b32, 16)]
        sc1 = sc_v[pl.ds(b32 + 16, 16)]
        copies = []
        for i in range(_CH):
            sh = (sh0, sh1)[i // 16][i % 16]
            sc = (sc0, sc1)[i // 16][i % 16]
            copies.append(pltpu.async_copy(
                wh_hbm.at[pl.ds(pl.multiple_of(sh, 8), 8)],
                hot_b.at[pl.ds(i * 8, 8)], sem))
            copies.append(pltpu.async_copy(
                whash_hbm.at[pl.ds(pl.multiple_of(sc, 8), 8)],
                cold_b.at[pl.ds(i * 8, 8)], sem))
        for c in copies:
            c.wait()

        rh0 = rh_v[pl.ds(b32, 16)]
        rh1 = rh_v[pl.ds(b32 + 16, 16)]
        rc0 = rc_v[pl.ds(b32, 16)]
        rc1 = rc_v[pl.ds(b32 + 16, 16)]
        m0 = maskf_v[pl.ds(b32, 16)]
        m1 = maskf_v[pl.ds(b32 + 16, 16)]
        for i in range(_CH):
            rh = (rh0, rh1)[i // 16][i % 16]
            rc = (rc0, rc1)[i // 16][i % 16]
            m = (m0, m1)[i // 16][i % 16]
            orow = ch * 8 + (i >> 2)
            ocol = (i & 3) * 32
            for c0 in (0, 16):
                h = hot_b[i * 8 + rh, pl.ds(c0, 16)]
                g = cold_b[i * 8 + rc, pl.ds(c0, 16)]
                out_b[orow, pl.ds(ocol + c0, 16)] = g + m * (h - g)
        return 0

    lax.fori_loop(0, _NCH, chunk, 0)
    pltpu.sync_copy(out_b, out_hbm.at[pl.ds(wid * 128, 128)])


@jax.jit
def _run(inp, wh, whash):
    mesh = plsc.VectorSubcoreMesh(core_axis_name="c", subcore_axis_name="s")
    f = pl.kernel(
        _sc_body,
        out_type=jax.ShapeDtypeStruct((BATCH // 4, 128), jnp.float32),
        mesh=mesh,
        compiler_params=pltpu.CompilerParams(use_tc_tiling_on_sc=True),
        scratch_types=[
            pltpu.VMEM((_BPW,), jnp.int32),
            pltpu.VMEM((_BPW,), jnp.int32),
            pltpu.VMEM((_BPW,), jnp.int32),
            pltpu.VMEM((_BPW,), jnp.int32),
            pltpu.VMEM((_BPW,), jnp.int32),
            pltpu.VMEM((_BPW,), jnp.float32),
            pltpu.VMEM((_CH * 8, 32), jnp.float32),
            pltpu.VMEM((_CH * 8, 32), jnp.float32),
            pltpu.VMEM((128, 128), jnp.float32),
            pltpu.SemaphoreType.DMA,
        ],
    )
    return f(inp, wh, whash)


def kernel(input, offsets, weight_h, weight_hash):
    del offsets
    out = _run(input.astype(jnp.int32), weight_h, weight_hash)
    return out.reshape(BATCH, EMB_DIM)


# speedup vs baseline: 6.3486x; 1.0281x over previous
import jax
import jax.numpy as jnp
from jax import lax
from jax.experimental import pallas as pl
from jax.experimental.pallas import tpu as pltpu
from jax.experimental.pallas import tpu_sc as plsc

HOTN = 32768
HASH_SIZE = 500000
EMB_DIM = 32
BATCH = 16384

_NC = 2
_NS = 16
_NW = _NC * _NS
_BPW = BATCH // _NW
_NVEC = _BPW // 16
_CH = 16
_NCH = _BPW // _CH


def _sc_body(inp_hbm, wh_hbm, whash_hbm, out_hbm,
             raw_v, sh_v, sc_v, rh_v, rc_v, maskf_v,
             hot_b, cold_b, out_b, sem):
    wid = lax.axis_index("s") * _NC + lax.axis_index("c")
    base = wid * _BPW

    pltpu.sync_copy(inp_hbm.at[pl.ds(base, _BPW)], raw_v)

    for i in range(_NVEC):
        v = raw_v[pl.ds(i * 16, 16)]
        s = (v & 31) + ((v >> 5) & 31) + ((v >> 10) & 31) + ((v >> 15) & 31)
        s = (s & 31) + (s >> 5)
        hot = jnp.logical_or(s == 0, s == 31)
        maskf_v[pl.ds(i * 16, 16)] = jnp.where(hot, 1.0, 0.0).astype(jnp.float32)
        rh = v & (HOTN - 1)
        rc = jnp.where(v >= HASH_SIZE, v - HASH_SIZE, v)
        sh_v[pl.ds(i * 16, 16)] = rh & ~7
        sc_v[pl.ds(i * 16, 16)] = rc & ~7
        rh_v[pl.ds(i * 16, 16)] = rh & 7
        rc_v[pl.ds(i * 16, 16)] = rc & 7

    def fire(ch1, p):
        sh16 = sh_v[pl.ds(ch1 * _CH, 16)]
        sc16 = sc_v[pl.ds(ch1 * _CH, 16)]
        for i in range(_CH):
            pltpu.async_copy(
                wh_hbm.at[pl.ds(pl.multiple_of(sh16[i], 8), 8)],
                hot_b.at[p, pl.ds(i * 8, 8)], sem.at[p])
            pltpu.async_copy(
                whash_hbm.at[pl.ds(pl.multiple_of(sc16[i], 8), 8)],
                cold_b.at[p, pl.ds(i * 8, 8)], sem.at[p])

    def drain(p):
        pltpu.make_async_copy(
            wh_hbm.at[pl.ds(0, _CH * 8)], hot_b.at[p], sem.at[p]).wait()
        pltpu.make_async_copy(
            wh_hbm.at[pl.ds(0, _CH * 8)], cold_b.at[p], sem.at[p]).wait()

    def blend(ch, p):
        rh16 = rh_v[pl.ds(ch * _CH, 16)]
        rc16 = rc_v[pl.ds(ch * _CH, 16)]
        m16 = maskf_v[pl.ds(ch * _CH, 16)]
        for i in range(_CH):
            rh = rh16[i]
            rc = rc16[i]
            m = m16[i]
            orow = ch * 4 + (i >> 2)
            ocol = (i & 3) * 32
            for c0 in (0, 16):
                h = hot_b[p, i * 8 + rh, pl.ds(c0, 16)]
                g = cold_b[p, i * 8 + rc, pl.ds(c0, 16)]
                out_b[orow, pl.ds(ocol + c0, 16)] = g + m * (h - g)

    fire(0, 0)

    def chunk(ch, _):
        for p in (0, 1):
            @pl.when((ch & 1) == p)
            def _():
                @pl.when(ch + 1 < _NCH)
                def _():
                    fire(ch + 1, 1 - p)
                drain(p)
                blend(ch, p)
        return 0

    lax.fori_loop(0, _NCH, chunk, 0)
    pltpu.sync_copy(out_b, out_hbm.at[pl.ds(wid * 128, 128)])


@jax.jit
def _run(inp, wh, whash):
    mesh = plsc.VectorSubcoreMesh(core_axis_name="c", subcore_axis_name="s")
    f = pl.kernel(
        _sc_body,
        out_type=jax.ShapeDtypeStruct((BATCH // 4, 128), jnp.float32),
        mesh=mesh,
        compiler_params=pltpu.CompilerParams(use_tc_tiling_on_sc=True),
        scratch_types=[
            pltpu.VMEM((_BPW,), jnp.int32),
            pltpu.VMEM((_BPW,), jnp.int32),
            pltpu.VMEM((_BPW,), jnp.int32),
            pltpu.VMEM((_BPW,), jnp.int32),
            pltpu.VMEM((_BPW,), jnp.int32),
            pltpu.VMEM((_BPW,), jnp.float32),
            pltpu.VMEM((2, _CH * 8, 32), jnp.float32),
            pltpu.VMEM((2, _CH * 8, 32), jnp.float32),
            pltpu.VMEM((128, 128), jnp.float32),
            pltpu.SemaphoreType.DMA((2,)),
        ],
    )
    return f(inp, wh, whash)


def kernel(input, offsets, weight_h, weight_hash):
    del offsets
    out = _run(input.astype(jnp.int32), weight_h, weight_hash)
    return out.reshape(BATCH, EMB_DIM)


# speedup vs baseline: 8.8085x; 1.3875x over previous
import jax
import jax.numpy as jnp
from jax import lax
from jax.experimental import pallas as pl
from jax.experimental.pallas import tpu as pltpu
from jax.experimental.pallas import tpu_sc as plsc

HOTN = 32768
HASH_SIZE = 500000
EMB_DIM = 32
BATCH = 16384

_NC = 2
_NS = 16
_NW = _NC * _NS
_BPW = BATCH // _NW
_NVEC = _BPW // 16
_CH = 16
_NCH = _BPW // _CH


def _sc_body(inp_hbm, wh_hbm, whash_hbm, out_hbm,
             raw_v, sh_v, sc_v, rh_v, rc_v, maskf_v,
             hot_b, cold_b, out_b, sem):
    wid = lax.axis_index("s") * _NC + lax.axis_index("c")
    base = wid * _BPW

    pltpu.sync_copy(inp_hbm.at[pl.ds(base, _BPW)], raw_v)

    for i in range(_NVEC):
        v = raw_v[pl.ds(i * 16, 16)]
        s = (v & 31) + ((v >> 5) & 31) + ((v >> 10) & 31) + ((v >> 15) & 31)
        s = (s & 31) + (s >> 5)
        hot = jnp.logical_or(s == 0, s == 31)
        maskf_v[pl.ds(i * 16, 16)] = jnp.where(hot, 1.0, 0.0).astype(jnp.float32)
        rh = v & (HOTN - 1)
        rc = jnp.where(v >= HASH_SIZE, v - HASH_SIZE, v)
        sh_v[pl.ds(i * 16, 16)] = rh >> 3
        sc_v[pl.ds(i * 16, 16)] = rc >> 3
        rh_v[pl.ds(i * 16, 16)] = rh & 7
        rc_v[pl.ds(i * 16, 16)] = rc & 7

    def fire(ch1, p):
        sh16 = sh_v[pl.ds(ch1 * _CH, 16)]
        sc16 = sc_v[pl.ds(ch1 * _CH, 16)]
        for i in range(_CH):
            pltpu.async_copy(wh_hbm.at[sh16[i]], hot_b.at[p, i], sem.at[p])
            pltpu.async_copy(whash_hbm.at[sc16[i]], cold_b.at[p, i], sem.at[p])

    def drain(p):
        pltpu.make_async_copy(
            wh_hbm.at[pl.ds(0, _CH)], hot_b.at[p], sem.at[p]).wait()
        pltpu.make_async_copy(
            wh_hbm.at[pl.ds(0, _CH)], cold_b.at[p], sem.at[p]).wait()

    def blend(ch, p):
        rh16 = rh_v[pl.ds(ch * _CH, 16)]
        rc16 = rc_v[pl.ds(ch * _CH, 16)]
        m16 = maskf_v[pl.ds(ch * _CH, 16)]
        for i in range(_CH):
            rh = rh16[i]
            rc = rc16[i]
            m = m16[i]
            orow = ch * 4 + (i >> 2)
            ocol = (i & 3) * 32
            for c0 in (0, 16):
                h = hot_b[p, i, rh, pl.ds(c0, 16)]
                g = cold_b[p, i, rc, pl.ds(c0, 16)]
                out_b[orow, pl.ds(ocol + c0, 16)] = g + m * (h - g)

    fire(0, 0)

    def chunk(ch, _):
        for p in (0, 1):
            @pl.when((ch & 1) == p)
            def _():
                @pl.when(ch + 1 < _NCH)
                def _():
                    fire(ch + 1, 1 - p)
                drain(p)
                blend(ch, p)
        return 0

    lax.fori_loop(0, _NCH, chunk, 0)
    pltpu.sync_copy(out_b, out_hbm.at[pl.ds(wid * 128, 128)])


@jax.jit
def _run(inp, wh, whash):
    mesh = plsc.VectorSubcoreMesh(core_axis_name="c", subcore_axis_name="s")
    f = pl.kernel(
        _sc_body,
        out_type=jax.ShapeDtypeStruct((BATCH // 4, 128), jnp.float32),
        mesh=mesh,
        compiler_params=pltpu.CompilerParams(use_tc_tiling_on_sc=True),
        scratch_types=[
            pltpu.VMEM((_BPW,), jnp.int32),
            pltpu.VMEM((_BPW,), jnp.int32),
            pltpu.VMEM((_BPW,), jnp.int32),
            pltpu.VMEM((_BPW,), jnp.int32),
            pltpu.VMEM((_BPW,), jnp.int32),
            pltpu.VMEM((_BPW,), jnp.float32),
            pltpu.VMEM((2, _CH, 8, 32), jnp.float32),
            pltpu.VMEM((2, _CH, 8, 32), jnp.float32),
            pltpu.VMEM((128, 128), jnp.float32),
            pltpu.SemaphoreType.DMA((2,)),
        ],
    )
    return f(inp, wh, whash)


def kernel(input, offsets, weight_h, weight_hash):
    del offsets
    wh = weight_h.reshape(HOTN // 8, 8, EMB_DIM)
    whash = weight_hash.reshape(HASH_SIZE // 8, 8, EMB_DIM)
    out = _run(input.astype(jnp.int32), wh, whash)
    return out.reshape(BATCH, EMB_DIM)


# speedup vs baseline: 10.4952x; 1.1915x over previous
import jax
import jax.numpy as jnp
from jax import lax
from jax.experimental import pallas as pl
from jax.experimental.pallas import tpu as pltpu
from jax.experimental.pallas import tpu_sc as plsc

HOTN = 32768
HASH_SIZE = 500000
EMB_DIM = 32
BATCH = 16384

_NC = 2
_NS = 16
_NW = _NC * _NS
_BPW = BATCH // _NW
_NVEC = _BPW // 16
_CH = 32
_NCH = _BPW // _CH


def _sc_body(inp_hbm, wh_hbm, whash_hbm, out_hbm,
             raw_v, sg_v, sr_v, m_v, grp_b, out_b, sem):
    wid = lax.axis_index("s") * _NC + lax.axis_index("c")
    base = wid * _BPW

    pltpu.sync_copy(inp_hbm.at[pl.ds(base, _BPW)], raw_v)

    for i in range(_NVEC):
        v = raw_v[pl.ds(i * 16, 16)]
        s = (v & 31) + ((v >> 5) & 31) + ((v >> 10) & 31) + ((v >> 15) & 31)
        s = (s & 31) + (s >> 5)
        hot = jnp.logical_or(s == 0, s == 31)
        rh = v & (HOTN - 1)
        rc = jnp.where(v >= HASH_SIZE, v - HASH_SIZE, v)
        r = jnp.where(hot, rh, rc)
        m_v[pl.ds(i * 16, 16)] = jnp.where(hot, 1, 0).astype(jnp.int32)
        sg_v[pl.ds(i * 16, 16)] = r >> 3
        sr_v[pl.ds(i * 16, 16)] = r & 7

    def fire(ch1, p):
        for h in range(_CH // 16):
            sg16 = sg_v[pl.ds(ch1 * _CH + h * 16, 16)]
            m16 = m_v[pl.ds(ch1 * _CH + h * 16, 16)]
            for j in range(16):
                i = h * 16 + j
                g = sg16[j]
                hotf = m16[j]

                @pl.when(hotf == 1)
                def _():
                    pltpu.async_copy(wh_hbm.at[g], grp_b.at[p, i], sem.at[p])

                @pl.when(hotf == 0)
                def _():
                    pltpu.async_copy(whash_hbm.at[g], grp_b.at[p, i],
                                     sem.at[p])

    def drain(p):
        pltpu.make_async_copy(
            whash_hbm.at[pl.ds(0, _CH)], grp_b.at[p], sem.at[p]).wait()

    def extract(ch, p):
        for h in range(_CH // 16):
            sr16 = sr_v[pl.ds(ch * _CH + h * 16, 16)]
            for j in range(16):
                i = h * 16 + j
                r = sr16[j]
                orow = ch * 8 + (i >> 2)
                ocol = (i & 3) * 32
                for c0 in (0, 16):
                    out_b[orow, pl.ds(ocol + c0, 16)] = \
                        grp_b[p, i, r, pl.ds(c0, 16)]

    fire(0, 0)

    def chunk(ch, _):
        for p in (0, 1):
            @pl.when((ch & 1) == p)
            def _():
                @pl.when(ch + 1 < _NCH)
                def _():
                    fire(ch + 1, 1 - p)
                drain(p)
                extract(ch, p)
        return 0

    lax.fori_loop(0, _NCH, chunk, 0)
    pltpu.sync_copy(out_b, out_hbm.at[pl.ds(wid * 128, 128)])


@jax.jit
def _run(inp, wh, whash):
    mesh = plsc.VectorSubcoreMesh(core_axis_name="c", subcore_axis_name="s")
    f = pl.kernel(
        _sc_body,
        out_type=jax.ShapeDtypeStruct((BATCH // 4, 128), jnp.float32),
        mesh=mesh,
        compiler_params=pltpu.CompilerParams(use_tc_tiling_on_sc=True),
        scratch_types=[
            pltpu.VMEM((_BPW,), jnp.int32),
            pltpu.VMEM((_BPW,), jnp.int32),
            pltpu.VMEM((_BPW,), jnp.int32),
            pltpu.VMEM((_BPW,), jnp.int32),
            pltpu.VMEM((2, _CH, 8, 32), jnp.float32),
            pltpu.VMEM((128, 128), jnp.float32),
            pltpu.SemaphoreType.DMA((2,)),
        ],
    )
    return f(inp, wh, whash)


def kernel(input, offsets, weight_h, weight_hash):
    del offsets
    wh = weight_h.reshape(HOTN // 8, 8, EMB_DIM)
    whash = weight_hash.reshape(HASH_SIZE // 8, 8, EMB_DIM)
    out = _run(input.astype(jnp.int32), wh, whash)
    return out.reshape(BATCH, EMB_DIM)
